# baseline (speedup 1.0000x reference)
import functools

import jax
import jax.numpy as jnp
from jax import lax
from jax.experimental import pallas as pl
from jax.experimental.pallas import tpu as pltpu
from jax.experimental.pallas import tpu_sc as plsc

NW = 32
CH = 128


def _gather_call(b, l, d2):
    bw = b // NW
    hw = bw // 2
    mesh = plsc.VectorSubcoreMesh(core_axis_name="c", subcore_axis_name="s")

    lp = 256

    @functools.partial(
        pl.kernel,
        mesh=mesh,
        out_type=jax.ShapeDtypeStruct((b, l, d2), jnp.float32),
        scratch_types=[
            pltpu.VMEM((hw, lp), jnp.int32),
            pltpu.VMEM((l, d2), jnp.float32),
            pltpu.VMEM((l, d2), jnp.float32),
            pltpu.VMEM((l, d2), jnp.float32),
            pltpu.VMEM((l, d2), jnp.float32),
            pltpu.SemaphoreType.DMA,
            pltpu.SemaphoreType.DMA,
            pltpu.SemaphoreType.DMA,
            pltpu.SemaphoreType.DMA,
            pltpu.SemaphoreType.DMA,
            pltpu.SemaphoreType.DMA,
            pltpu.SemaphoreType.DMA,
            pltpu.SemaphoreType.DMA,
        ],
    )
    def k(x_hbm, tpad_hbm, out_hbm, idxh, rows0, rows1, rows2, rows3,
          sg0, sg1, sg2, sg3, so0, so1, so2, so3):
        wid = lax.axis_index("s") * 2 + lax.axis_index("c")
        blo = wid * bw
        rows = (rows0, rows1, rows2, rows3)
        sg = (sg0, sg1, sg2, sg3)
        so = (so0, so1, so2, so3)

        def stage_idx(half):
            pltpu.sync_copy(x_hbm.at[pl.ds(blo + half * hw, hw)], idxh)

        def fire_gathers(s, r):
            q = r & (hw - 1)
            pltpu.async_copy(
                tpad_hbm.at[idxh.at[q, pl.ds(0, CH)]],
                rows[s].at[pl.ds(0, CH)],
                sg[s],
            )
            pltpu.async_copy(
                tpad_hbm.at[idxh.at[q, pl.ds(CH, l - CH)]],
                rows[s].at[pl.ds(CH, l - CH)],
                sg[s],
            )

        def wait_gathers(s):
            pltpu.make_async_copy(
                tpad_hbm.at[pl.ds(0, l)], rows[s], sg[s]
            ).wait()

        def fire_out(s, r):
            pltpu.async_copy(rows[s], out_hbm.at[blo + r], so[s])

        def wait_out(s):
            pltpu.make_async_copy(rows[s], out_hbm.at[blo], so[s]).wait()

        def step(r, s):
            s3 = (s + 3) % 4
            wait_out(s3)
            fire_gathers(s3, r + 3)
            wait_gathers(s)
            fire_out(s, r)

        stage_idx(0)
        for r0 in range(4):
            fire_gathers(r0 % 4, r0)
        wait_gathers(0)
        fire_out(0, 0)

        def quad_body1(t, carry):
            r = 1 + 4 * t
            for kk in range(4):
                step(r + kk, (1 + kk) % 4)
            return carry

        lax.fori_loop(0, (hw - 4) // 4, quad_body1, 0)

        for r in range(hw - 3, hw):
            s = r % 4
            wait_gathers(s)
            fire_out(s, r)
        stage_idx(1)
        for r in range(hw, hw + 3):
            s = r % 4
            wait_out(s)
            fire_gathers(s, r)

        def quad_body2(t, carry):
            r = hw + 4 * t
            for kk in range(4):
                step(r + kk, (hw + kk) % 4)
            return carry

        lax.fori_loop(0, (bw - hw - 4) // 4, quad_body2, 0)

        step(bw - 4, (bw - 4) % 4)
        for r in range(bw - 3, bw):
            s = r % 4
            wait_gathers(s)
            fire_out(s, r)
        for s in range(4):
            wait_out(s)

    return k


def kernel(x, table):
    b, l = x.shape
    d = table.shape[1]
    tpad = jnp.pad(table, ((0, 0), (0, d)))
    xpad = jnp.pad(x.astype(jnp.int32), ((0, 0), (0, 256 - l)))
    out1 = _gather_call(b, l, 2 * d)(xpad, tpad)
    return out1[..., :d]
